# baseline (speedup 1.0000x reference)
import functools

import jax
import jax.numpy as jnp
from jax import lax
from jax.experimental import pallas as pl
from jax.experimental.pallas import tpu as pltpu
from jax.experimental.pallas import tpu_sc as plsc

N_NODE = 10000
E = 320000
D = 128
QPAD = 10240

NC = 2
NS = 16
NW = NC * NS
K = 128
ECH = E // K
QCH = QPAD // K
WB_TILES = 10
WB_ROWS = N_NODE // WB_TILES

EB = 2000
F32 = jnp.float32


def _qr_body(qrel_hbm, rela_hbm, qr_out, idx_v, rows_v, sem):
    cid = lax.axis_index("c")
    sid = lax.axis_index("s")
    wid = sid * NC + cid

    def chunk(ci, carry):
        off = ci * K
        pltpu.sync_copy(qrel_hbm.at[pl.ds(off, K)], idx_v)
        pltpu.async_copy(rela_hbm.at[idx_v], rows_v, sem).wait()
        pltpu.sync_copy(rows_v, qr_out.at[pl.ds(off, K)])
        return carry

    lax.fori_loop((wid * QCH) // NW, ((wid + 1) * QCH) // NW, chunk, 0)


_qr_gather = functools.partial(
    pl.kernel,
    out_type=jax.ShapeDtypeStruct((QPAD, D), F32),
    mesh=plsc.VectorSubcoreMesh(core_axis_name="c", subcore_axis_name="s"),
    scratch_types=[
        pltpu.VMEM((K,), jnp.int32),
        pltpu.VMEM((K, D), F32),
        pltpu.SemaphoreType.DMA,
    ],
)(_qr_body)


def _gather_body(sub_hbm, e2_hbm, ridx_hbm, hidden_hbm, rela_hbm, qr_hbm,
                 hs_out, hr_out, hqr_out,
                 idx1, idx2, idx3, rows1, rows2, rows3, sem, semi):
    cid = lax.axis_index("c")
    sid = lax.axis_index("s")
    wid = sid * NC + cid

    def chunk(ci, carry):
        off = ci * K
        i1 = pltpu.async_copy(sub_hbm.at[pl.ds(off, K)], idx1, semi)
        i2 = pltpu.async_copy(e2_hbm.at[pl.ds(off, K)], idx2, semi)
        i3 = pltpu.async_copy(ridx_hbm.at[pl.ds(off, K)], idx3, semi)
        i1.wait()
        i2.wait()
        i3.wait()
        c1 = pltpu.async_copy(hidden_hbm.at[idx1], rows1, sem)
        c2 = pltpu.async_copy(rela_hbm.at[idx2], rows2, sem)
        c3 = pltpu.async_copy(qr_hbm.at[idx3], rows3, sem)
        c1.wait()
        pltpu.sync_copy(rows1, hs_out.at[pl.ds(off, K)])
        c2.wait()
        pltpu.sync_copy(rows2, hr_out.at[pl.ds(off, K)])
        c3.wait()
        pltpu.sync_copy(rows3, hqr_out.at[pl.ds(off, K)])
        return carry

    lax.fori_loop((wid * ECH) // NW, ((wid + 1) * ECH) // NW, chunk, 0)


_gather = functools.partial(
    pl.kernel,
    out_type=[jax.ShapeDtypeStruct((E, D), F32)] * 3,
    mesh=plsc.VectorSubcoreMesh(core_axis_name="c", subcore_axis_name="s"),
    scratch_types=[
        pltpu.VMEM((K,), jnp.int32),
        pltpu.VMEM((K,), jnp.int32),
        pltpu.VMEM((K,), jnp.int32),
        pltpu.VMEM((K, D), F32),
        pltpu.VMEM((K, D), F32),
        pltpu.VMEM((K, D), F32),
        pltpu.SemaphoreType.DMA,
        pltpu.SemaphoreType.DMA,
    ],
)(_gather_body)


def _msg_scatter_body(up_hbm, obj_hbm, zeros_hbm, msg_part,
                      rows_v, idx_v, sem, msg_acc):
    cid = lax.axis_index("c")
    sid = lax.axis_index("s")
    wid = sid * NC + cid

    @pl.when(sid == 0)
    def _():
        pltpu.sync_copy(zeros_hbm, msg_acc)

    plsc.subcore_barrier()

    def chunk(ci, carry):
        off = ci * K
        pltpu.sync_copy(up_hbm.at[pl.ds(off, K)], rows_v)
        pltpu.sync_copy(obj_hbm.at[pl.ds(off, K)], idx_v)
        pltpu.sync_copy(rows_v, msg_acc.at[idx_v], add=True)
        return carry

    lax.fori_loop((wid * ECH) // NW, ((wid + 1) * ECH) // NW, chunk, 0)
    plsc.subcore_barrier()

    @pl.when(sid == 0)
    def _():
        pltpu.sync_copy(msg_acc, msg_part.at[cid])


_msg_scatter = functools.partial(
    pl.kernel,
    out_type=jax.ShapeDtypeStruct((NC, N_NODE, D), F32),
    mesh=plsc.VectorSubcoreMesh(core_axis_name="c", subcore_axis_name="s"),
    scratch_types=[
        pltpu.VMEM((K, D), F32),
        pltpu.VMEM((K,), jnp.int32),
        pltpu.SemaphoreType.DMA,
        pltpu.VMEM_SHARED((N_NODE, D), F32),
    ],
)(_msg_scatter_body)


def _deg_scatter_body(obj_hbm, zeros_hbm, ones_hbm, deg_part,
                      idx_v, ones_v, sem, deg_acc):
    cid = lax.axis_index("c")
    sid = lax.axis_index("s")
    wid = sid * NC + cid

    @pl.when(sid == 0)
    def _():
        pltpu.sync_copy(zeros_hbm, deg_acc)

    pltpu.sync_copy(ones_hbm, ones_v)
    plsc.subcore_barrier()

    def chunk(ci, carry):
        off = ci * K
        pltpu.sync_copy(obj_hbm.at[pl.ds(off, K)], idx_v)
        pltpu.sync_copy(ones_v, deg_acc.at[idx_v], add=True)
        return carry

    lax.fori_loop((wid * ECH) // NW, ((wid + 1) * ECH) // NW, chunk, 0)
    plsc.subcore_barrier()

    @pl.when(sid == 0)
    def _():
        pltpu.sync_copy(deg_acc, deg_part.at[cid])


_deg_scatter = functools.partial(
    pl.kernel,
    out_type=jax.ShapeDtypeStruct((NC, N_NODE, D), F32),
    mesh=plsc.VectorSubcoreMesh(core_axis_name="c", subcore_axis_name="s"),
    scratch_types=[
        pltpu.VMEM((K,), jnp.int32),
        pltpu.VMEM((K, D), F32),
        pltpu.SemaphoreType.DMA,
        pltpu.VMEM_SHARED((N_NODE, D), F32),
    ],
)(_deg_scatter_body)


def _dense_body(hs_ref, hr_ref, hqr_ref, gwr_ref, gwq_ref, gws_ref, gb_ref,
                ht1_ref, ht2_ref, htb_ref, ws_ref, wr_ref, wqr_ref, wqrb_ref,
                wa_ref, out_ref):
    hs = hs_ref[...]
    hr = hr_ref[...]
    hqr = hqr_ref[...]
    bf = jnp.bfloat16
    hsb = hs.astype(bf)
    hrb = hr.astype(bf)
    hqrb = hqr.astype(bf)
    gp = (jnp.dot(hrb, gwr_ref[...].astype(bf), preferred_element_type=F32)
          + jnp.dot(hqrb, gwq_ref[...].astype(bf), preferred_element_type=F32)
          + jnp.dot(hsb, gws_ref[...].astype(bf), preferred_element_type=F32)
          + gb_ref[...])
    g = jax.nn.sigmoid(gp)
    u = g[:, :D]
    r = g[:, D:]
    hc = jnp.tanh(jnp.dot(hrb, ht1_ref[...].astype(bf), preferred_element_type=F32)
                  + jnp.dot((r * hs).astype(bf), ht2_ref[...].astype(bf),
                            preferred_element_type=F32)
                  + htb_ref[...])
    ap = (jnp.dot(hsb, ws_ref[...].astype(bf), preferred_element_type=F32)
          + jnp.dot(hrb, wr_ref[...].astype(bf), preferred_element_type=F32)
          + jnp.dot(hqrb, wqr_ref[...].astype(bf), preferred_element_type=F32)
          + wqrb_ref[...])
    lr = jnp.where(ap >= 0.0, ap, 0.01 * ap)
    al = jnp.sum(lr * wa_ref[...], axis=1, keepdims=True)
    att = jax.nn.sigmoid(al)
    out_ref[...] = att * ((1.0 - u) * hs + u * hc)


def _edge_spec():
    return pl.BlockSpec((EB, D), lambda i: (i, 0))


def _w_spec(shape):
    return pl.BlockSpec(shape, lambda i: (0, 0))


_dense = pl.pallas_call(
    _dense_body,
    grid=(E // EB,),
    in_specs=[_edge_spec(), _edge_spec(), _edge_spec(),
              _w_spec((D, 2 * D)), _w_spec((D, 2 * D)), _w_spec((D, 2 * D)),
              _w_spec((1, 2 * D)),
              _w_spec((D, D)), _w_spec((D, D)), _w_spec((1, D)),
              _w_spec((D, D)), _w_spec((D, D)), _w_spec((D, D)),
              _w_spec((1, D)), _w_spec((1, D))],
    out_specs=_edge_spec(),
    out_shape=jax.ShapeDtypeStruct((E, D), F32),
)


def _final_body(msg_ref, deg_ref, wh_ref, out_ref):
    agg = msg_ref[0] + msg_ref[1]
    deg = deg_ref[0, :, 0:1] + deg_ref[1, :, 0:1]
    scale = lax.rsqrt(deg + 1e-4)
    out_ref[...] = jnp.dot(agg * scale, wh_ref[...],
                           preferred_element_type=F32)


_FB = 1000
_final = pl.pallas_call(
    _final_body,
    grid=(N_NODE // _FB,),
    in_specs=[pl.BlockSpec((NC, _FB, D), lambda i: (0, i, 0)),
              pl.BlockSpec((NC, _FB, D), lambda i: (0, i, 0)),
              pl.BlockSpec((D, D), lambda i: (0, 0))],
    out_specs=pl.BlockSpec((_FB, D), lambda i: (i, 0)),
    out_shape=jax.ShapeDtypeStruct((N_NODE, D), F32),
)


def kernel(q_sub, q_rel, hidden, edges, n_node, rela_embed, Ws_W, Wr_W,
           Wqr_W, Wqr_b, walpha_W, gate_W, gate_b, ht_W, ht_b, Wh_W):
    del q_sub, n_node
    edges = edges.astype(jnp.int32)
    sub = edges[:, 4]
    obj = edges[:, 5]
    ridx = edges[:, 0]
    e2 = edges[:, 2]
    q_rel = jnp.pad(q_rel.astype(jnp.int32), (0, QPAD - q_rel.shape[0]))

    qr_embed = _qr_gather(q_rel, rela_embed)
    hs, hr, hqr = _gather(sub, e2, ridx, hidden, rela_embed, qr_embed)

    gwr = gate_W[:D]
    gwq = gate_W[D:2 * D]
    gws = gate_W[2 * D:]
    up_msg = _dense(hs, hr, hqr, gwr, gwq, gws, gate_b.reshape(1, 2 * D),
                    ht_W[:D], ht_W[D:], ht_b.reshape(1, D),
                    Ws_W, Wr_W, Wqr_W, Wqr_b.reshape(1, D),
                    walpha_W.reshape(1, D))

    zeros = jnp.zeros((N_NODE, D), F32)
    ones_blk = jnp.ones((K, D), F32)
    msg_part = _msg_scatter(up_msg, obj, zeros)
    deg_part = _deg_scatter(obj, zeros, ones_blk)

    return _final(msg_part, deg_part, Wh_W)


# speedup vs baseline: 1.0807x; 1.0807x over previous
import functools

import jax
import jax.numpy as jnp
from jax import lax
from jax.experimental import pallas as pl
from jax.experimental.pallas import tpu as pltpu
from jax.experimental.pallas import tpu_sc as plsc

N_NODE = 10000
E = 320000
D = 128
QPAD = 10240

NC = 2
NS = 16
NW = NC * NS
K = 128
ECH = E // K
QCH = QPAD // K
WB_TILES = 10
WB_ROWS = N_NODE // WB_TILES

EB = 2000
F32 = jnp.float32


def _qr_body(qrel_hbm, rela_hbm, qr_out, idx_v, rows_v, sem):
    cid = lax.axis_index("c")
    sid = lax.axis_index("s")
    wid = sid * NC + cid

    def chunk(ci, carry):
        off = ci * K
        pltpu.sync_copy(qrel_hbm.at[pl.ds(off, K)], idx_v)
        pltpu.async_copy(rela_hbm.at[idx_v], rows_v, sem).wait()
        pltpu.sync_copy(rows_v, qr_out.at[pl.ds(off, K)])
        return carry

    lax.fori_loop((wid * QCH) // NW, ((wid + 1) * QCH) // NW, chunk, 0)


_qr_gather = functools.partial(
    pl.kernel,
    out_type=jax.ShapeDtypeStruct((QPAD, D), F32),
    mesh=plsc.VectorSubcoreMesh(core_axis_name="c", subcore_axis_name="s"),
    scratch_types=[
        pltpu.VMEM((K,), jnp.int32),
        pltpu.VMEM((K, D), F32),
        pltpu.SemaphoreType.DMA,
    ],
)(_qr_body)


def _gather_body(sub_hbm, e2_hbm, ridx_hbm, hidden_hbm, rela_hbm, qr_hbm,
                 hs_out, hr_out, hqr_out,
                 idx1, idx2, idx3, rows1, rows2, rows3, sem, semi):
    cid = lax.axis_index("c")
    sid = lax.axis_index("s")
    wid = sid * NC + cid

    def chunk(ci, carry):
        off = ci * K
        i1 = pltpu.async_copy(sub_hbm.at[pl.ds(off, K)], idx1, semi)
        i2 = pltpu.async_copy(e2_hbm.at[pl.ds(off, K)], idx2, semi)
        i3 = pltpu.async_copy(ridx_hbm.at[pl.ds(off, K)], idx3, semi)
        i1.wait()
        i2.wait()
        i3.wait()
        c1 = pltpu.async_copy(hidden_hbm.at[idx1], rows1, sem)
        c2 = pltpu.async_copy(rela_hbm.at[idx2], rows2, sem)
        c3 = pltpu.async_copy(qr_hbm.at[idx3], rows3, sem)
        c1.wait()
        pltpu.sync_copy(rows1, hs_out.at[pl.ds(off, K)])
        c2.wait()
        pltpu.sync_copy(rows2, hr_out.at[pl.ds(off, K)])
        c3.wait()
        pltpu.sync_copy(rows3, hqr_out.at[pl.ds(off, K)])
        return carry

    lax.fori_loop((wid * ECH) // NW, ((wid + 1) * ECH) // NW, chunk, 0)


_gather = functools.partial(
    pl.kernel,
    out_type=[jax.ShapeDtypeStruct((E, D), F32)] * 3,
    mesh=plsc.VectorSubcoreMesh(core_axis_name="c", subcore_axis_name="s"),
    scratch_types=[
        pltpu.VMEM((K,), jnp.int32),
        pltpu.VMEM((K,), jnp.int32),
        pltpu.VMEM((K,), jnp.int32),
        pltpu.VMEM((K, D), F32),
        pltpu.VMEM((K, D), F32),
        pltpu.VMEM((K, D), F32),
        pltpu.SemaphoreType.DMA,
        pltpu.SemaphoreType.DMA,
    ],
)(_gather_body)


def _msg_scatter_body(up_hbm, obj_hbm, zeros_hbm, msg_part,
                      rows_v, idx_v, sem, msg_acc):
    cid = lax.axis_index("c")
    sid = lax.axis_index("s")
    wid = sid * NC + cid

    @pl.when(sid == 0)
    def _():
        pltpu.sync_copy(zeros_hbm, msg_acc)

    plsc.subcore_barrier()

    def chunk(ci, carry):
        off = ci * K
        pltpu.sync_copy(up_hbm.at[pl.ds(off, K)], rows_v)
        pltpu.sync_copy(obj_hbm.at[pl.ds(off, K)], idx_v)
        pltpu.sync_copy(rows_v, msg_acc.at[idx_v], add=True)
        return carry

    lax.fori_loop((wid * ECH) // NW, ((wid + 1) * ECH) // NW, chunk, 0)
    plsc.subcore_barrier()

    @pl.when(sid == 0)
    def _():
        pltpu.sync_copy(msg_acc, msg_part.at[cid])


_msg_scatter = functools.partial(
    pl.kernel,
    out_type=jax.ShapeDtypeStruct((NC, N_NODE, D), F32),
    mesh=plsc.VectorSubcoreMesh(core_axis_name="c", subcore_axis_name="s"),
    scratch_types=[
        pltpu.VMEM((K, D), F32),
        pltpu.VMEM((K,), jnp.int32),
        pltpu.SemaphoreType.DMA,
        pltpu.VMEM_SHARED((N_NODE, D), F32),
    ],
)(_msg_scatter_body)


def _deg_scatter_body(obj_hbm, zeros_hbm, ones_hbm, deg_part,
                      idx_v, ones_v, sem, deg_acc):
    cid = lax.axis_index("c")
    sid = lax.axis_index("s")
    wid = sid * NC + cid

    @pl.when(sid == 0)
    def _():
        pltpu.sync_copy(zeros_hbm, deg_acc)

    pltpu.sync_copy(ones_hbm, ones_v)
    plsc.subcore_barrier()

    def chunk(ci, carry):
        off = ci * K
        pltpu.sync_copy(obj_hbm.at[pl.ds(off, K)], idx_v)
        pltpu.sync_copy(ones_v, deg_acc.at[idx_v], add=True)
        return carry

    lax.fori_loop((wid * ECH) // NW, ((wid + 1) * ECH) // NW, chunk, 0)
    plsc.subcore_barrier()

    @pl.when(sid == 0)
    def _():
        pltpu.sync_copy(deg_acc, deg_part.at[cid])


_deg_scatter = functools.partial(
    pl.kernel,
    out_type=jax.ShapeDtypeStruct((NC, N_NODE, D), F32),
    mesh=plsc.VectorSubcoreMesh(core_axis_name="c", subcore_axis_name="s"),
    scratch_types=[
        pltpu.VMEM((K,), jnp.int32),
        pltpu.VMEM((K, D), F32),
        pltpu.SemaphoreType.DMA,
        pltpu.VMEM_SHARED((N_NODE, D), F32),
    ],
)(_deg_scatter_body)


def _dense_body(hs_ref, hr_ref, hqr_ref, gwr_ref, gwq_ref, gws_ref, gb_ref,
                ht1_ref, ht2_ref, htb_ref, ws_ref, wr_ref, wqr_ref, wqrb_ref,
                wa_ref, out_ref):
    hs = hs_ref[...]
    hr = hr_ref[...]
    hqr = hqr_ref[...]
    gp = (jnp.dot(hr, gwr_ref[...], preferred_element_type=F32)
          + jnp.dot(hqr, gwq_ref[...], preferred_element_type=F32)
          + jnp.dot(hs, gws_ref[...], preferred_element_type=F32)
          + gb_ref[...])
    g = jax.nn.sigmoid(gp)
    u = g[:, :D]
    r = g[:, D:]
    hc = jnp.tanh(jnp.dot(hr, ht1_ref[...], preferred_element_type=F32)
                  + jnp.dot(r * hs, ht2_ref[...], preferred_element_type=F32)
                  + htb_ref[...])
    ap = (jnp.dot(hs, ws_ref[...], preferred_element_type=F32)
          + jnp.dot(hr, wr_ref[...], preferred_element_type=F32)
          + jnp.dot(hqr, wqr_ref[...], preferred_element_type=F32)
          + wqrb_ref[...])
    lr = jnp.where(ap >= 0.0, ap, 0.01 * ap)
    al = jnp.sum(lr * wa_ref[...], axis=1, keepdims=True)
    att = jax.nn.sigmoid(al)
    out_ref[...] = att * ((1.0 - u) * hs + u * hc)


def _edge_spec():
    return pl.BlockSpec((EB, D), lambda i: (i, 0))


def _w_spec(shape):
    return pl.BlockSpec(shape, lambda i: (0, 0))


_dense = pl.pallas_call(
    _dense_body,
    grid=(E // EB,),
    in_specs=[_edge_spec(), _edge_spec(), _edge_spec(),
              _w_spec((D, 2 * D)), _w_spec((D, 2 * D)), _w_spec((D, 2 * D)),
              _w_spec((1, 2 * D)),
              _w_spec((D, D)), _w_spec((D, D)), _w_spec((1, D)),
              _w_spec((D, D)), _w_spec((D, D)), _w_spec((D, D)),
              _w_spec((1, D)), _w_spec((1, D))],
    out_specs=_edge_spec(),
    out_shape=jax.ShapeDtypeStruct((E, D), F32),
)


def _final_body(msg_ref, deg_ref, wh_ref, out_ref):
    agg = msg_ref[0] + msg_ref[1]
    deg = deg_ref[0, :, 0:1] + deg_ref[1, :, 0:1]
    scale = lax.rsqrt(deg + 1e-4)
    out_ref[...] = jnp.dot(agg * scale, wh_ref[...],
                           preferred_element_type=F32)


_FB = 1000
_final = pl.pallas_call(
    _final_body,
    grid=(N_NODE // _FB,),
    in_specs=[pl.BlockSpec((NC, _FB, D), lambda i: (0, i, 0)),
              pl.BlockSpec((NC, _FB, D), lambda i: (0, i, 0)),
              pl.BlockSpec((D, D), lambda i: (0, 0))],
    out_specs=pl.BlockSpec((_FB, D), lambda i: (i, 0)),
    out_shape=jax.ShapeDtypeStruct((N_NODE, D), F32),
)


def kernel(q_sub, q_rel, hidden, edges, n_node, rela_embed, Ws_W, Wr_W,
           Wqr_W, Wqr_b, walpha_W, gate_W, gate_b, ht_W, ht_b, Wh_W):
    del q_sub, n_node
    edges = edges.astype(jnp.int32)
    sub = edges[:, 4]
    obj = edges[:, 5]
    ridx = edges[:, 0]
    e2 = edges[:, 2]
    q_rel = jnp.pad(q_rel.astype(jnp.int32), (0, QPAD - q_rel.shape[0]))

    qr_embed = _qr_gather(q_rel, rela_embed)
    hs, hr, hqr = _gather(sub, e2, ridx, hidden, rela_embed, qr_embed)

    gwr = gate_W[:D]
    gwq = gate_W[D:2 * D]
    gws = gate_W[2 * D:]
    up_msg = _dense(hs, hr, hqr, gwr, gwq, gws, gate_b.reshape(1, 2 * D),
                    ht_W[:D], ht_W[D:], ht_b.reshape(1, D),
                    Ws_W, Wr_W, Wqr_W, Wqr_b.reshape(1, D),
                    walpha_W.reshape(1, D))

    zeros = jnp.zeros((N_NODE, D), F32)
    ones_blk = jnp.ones((K, D), F32)
    msg_part = _msg_scatter(up_msg, obj, zeros)
    deg_part = _deg_scatter(obj, zeros, ones_blk)

    return _final(msg_part, deg_part, Wh_W)
